# Initial kernel scaffold; baseline (speedup 1.0000x reference)
import jax
import jax.numpy as jnp
from jax.experimental import pallas as pl


def kernel(unknown, known, unknow_feats, known_feats, image_features, new_vis, V2R, P2, image_shape, mlp_params, f3d_pre, f3d_mlp, f2d_mlp, f2d_conv):
    raise NotImplementedError("write your pallas kernel here")


# speedup vs baseline: 6.7876x; 6.7876x over previous
import jax
import jax.numpy as jnp
from jax.experimental import pallas as pl

F32 = jnp.float32


def _tile(R, cap=512):
    for t in (cap, 256, 128, 64, 32, 16, 8, 4, 2, 1):
        if t <= cap and R % t == 0:
            return t
    return 1


def _fused_mm(xs, Ws, affs, cnt=None):
    R = xs[0].shape[0]
    Cout = Ws[0].shape[1]
    TR = _tile(R)
    NT = R // TR
    nin = len(xs)
    have_cnt = cnt is not None

    def body(*refs):
        i = 0
        xr = refs[i:i + nin]; i += nin
        wr = refs[i:i + nin]; i += nin
        affr = []
        for a in affs:
            if a is not None:
                affr.append((refs[i], refs[i + 1])); i += 2
            else:
                affr.append(None)
        c_ref = None
        if have_cnt:
            c_ref = refs[i]; i += 1
        z_ref, st_ref = refs[i], refs[i + 1]
        acc = None
        for k in range(nin):
            x = xr[k][...]
            if affr[k] is not None:
                x = jnp.maximum(x * affr[k][0][...] + affr[k][1][...], 0.0)
            if k == 0 and c_ref is not None:
                x = x * (1.0 / (c_ref[...] + 1e-6))
            d = jnp.dot(x, wr[k][...], preferred_element_type=F32)
            acc = d if acc is None else acc + d
        z_ref[...] = acc

        @pl.when(pl.program_id(0) == 0)
        def _():
            st_ref[...] = jnp.zeros_like(st_ref)
        st_ref[0:1, :] += jnp.sum(acc, axis=0, keepdims=True)
        st_ref[1:2, :] += jnp.sum(acc * acc, axis=0, keepdims=True)

    ins, specs = [], []
    for x in xs:
        ins.append(x)
        specs.append(pl.BlockSpec((TR, x.shape[1]), lambda i: (i, 0)))
    for W in Ws:
        ins.append(W)
        specs.append(pl.BlockSpec(W.shape, lambda i: (0, 0)))
    for a in affs:
        if a is not None:
            for v in a:
                ins.append(v)
                specs.append(pl.BlockSpec((1, v.shape[1]), lambda i: (0, 0)))
    if have_cnt:
        ins.append(cnt)
        specs.append(pl.BlockSpec((TR, 1), lambda i: (i, 0)))

    z, st = pl.pallas_call(
        body,
        grid=(NT,),
        in_specs=specs,
        out_specs=[pl.BlockSpec((TR, Cout), lambda i: (i, 0)),
                   pl.BlockSpec((8, Cout), lambda i: (0, 0))],
        out_shape=[jax.ShapeDtypeStruct((R, Cout), F32),
                   jax.ShapeDtypeStruct((8, Cout), F32)],
    )(*ins)
    return z, st


def _bn_affine(st, g, b, R):
    mean = st[0] / R
    var = st[1] / R - mean * mean
    sc = g / jnp.sqrt(var + 1e-5)
    sh = b - mean * sc
    return sc.reshape(1, -1).astype(F32), sh.reshape(1, -1).astype(F32)


def _apply_affine_relu(z, sc, sh):
    R, C = z.shape
    TR = _tile(R)

    def body(z_ref, s_ref, h_ref, o_ref):
        o_ref[...] = jnp.maximum(z_ref[...] * s_ref[...] + h_ref[...], 0.0)

    return pl.pallas_call(
        body,
        grid=(R // TR,),
        in_specs=[pl.BlockSpec((TR, C), lambda i: (i, 0)),
                  pl.BlockSpec((1, C), lambda i: (0, 0)),
                  pl.BlockSpec((1, C), lambda i: (0, 0))],
        out_specs=pl.BlockSpec((TR, C), lambda i: (i, 0)),
        out_shape=jax.ShapeDtypeStruct((R, C), F32),
    )(z, sc, sh)


def _knn_interp(unknown, known_t, kf):
    B, n, _ = unknown.shape
    m = known_t.shape[2]
    C = kf.shape[2]
    TN = _tile(n, 256)

    def body(u_ref, k_ref, f_ref, o_ref):
        u = u_ref[0]
        k = k_ref[0]
        d2 = ((u[:, 0:1] - k[0:1, :]) ** 2
              + (u[:, 1:2] - k[1:2, :]) ** 2
              + (u[:, 2:3] - k[2:3, :]) ** 2)
        m1 = jnp.min(d2, axis=1, keepdims=True)
        d2a = jnp.where(d2 > m1, d2, jnp.inf)
        m2 = jnp.min(d2a, axis=1, keepdims=True)
        d2b = jnp.where(d2a > m2, d2a, jnp.inf)
        m3 = jnp.min(d2b, axis=1, keepdims=True)
        msk = d2 <= m3
        dist = jnp.sqrt(jnp.maximum(d2, 1e-12))
        recip = jnp.where(msk, 1.0 / (dist + 1e-8), 0.0)
        wgt = recip / jnp.sum(recip, axis=1, keepdims=True)
        o_ref[0] = jnp.dot(wgt, f_ref[0], preferred_element_type=F32)

    return pl.pallas_call(
        body,
        grid=(B, n // TN),
        in_specs=[pl.BlockSpec((1, TN, 3), lambda b, t: (b, t, 0)),
                  pl.BlockSpec((1, 3, m), lambda b, t: (b, 0, 0)),
                  pl.BlockSpec((1, m, C), lambda b, t: (b, 0, 0))],
        out_specs=pl.BlockSpec((1, TN, C), lambda b, t: (b, t, 0)),
        out_shape=jax.ShapeDtypeStruct((B, n, C), F32),
    )(unknown, known_t, kf)


def _pif_gather(proj, imf):
    B, n, _ = proj.shape
    P, C = imf.shape[1], imf.shape[2]
    TN = _tile(n)
    THW = _tile(P, 1920)
    NHW = P // THW

    def body(p_ref, im_ref, o_ref):
        k = pl.program_id(2)
        base = (k * THW).astype(F32)
        pj = p_ref[0]
        iot = jax.lax.broadcasted_iota(jnp.int32, (TN, THW), 1).astype(F32) + base
        S = (jnp.where(iot == pj[:, 0:1], pj[:, 4:5], 0.0)
             + jnp.where(iot == pj[:, 1:2], pj[:, 5:6], 0.0)
             + jnp.where(iot == pj[:, 2:3], pj[:, 6:7], 0.0)
             + jnp.where(iot == pj[:, 3:4], pj[:, 7:8], 0.0))
        S = S * pj[:, 10:11]
        acc = jnp.dot(S, im_ref[0], preferred_element_type=F32)

        @pl.when(k == 0)
        def _():
            o_ref[...] = jnp.zeros_like(o_ref)
        o_ref[0] += acc

    return pl.pallas_call(
        body,
        grid=(B, n // TN, NHW),
        in_specs=[pl.BlockSpec((1, TN, 16), lambda b, t, k: (b, t, 0)),
                  pl.BlockSpec((1, THW, C), lambda b, t, k: (b, k, 0))],
        out_specs=pl.BlockSpec((1, TN, C), lambda b, t, k: (b, t, 0)),
        out_shape=jax.ShapeDtypeStruct((B, n, C), F32),
    )(proj, imf)


def _scatter_grid(z2, sc2, sh2, projT, P):
    B, n, C = z2.shape
    TN = _tile(n)
    THW = _tile(P, 1920)

    def body(z_ref, s_ref, h_ref, pT_ref, g_ref, c_ref):
        t = pl.program_id(2)
        h = pl.program_id(1)
        nf = jnp.maximum(z_ref[0] * s_ref[...] + h_ref[...], 0.0)
        srow = pT_ref[0, 8:9, :]
        mrow = pT_ref[0, 9:10, :]
        base = (h * THW).astype(F32)
        iop = jax.lax.broadcasted_iota(jnp.int32, (THW, TN), 0).astype(F32) + base
        oh = jnp.where(iop == srow, mrow, 0.0)

        @pl.when(t == 0)
        def _():
            g_ref[...] = jnp.zeros_like(g_ref)
            c_ref[...] = jnp.zeros_like(c_ref)
        g_ref[0] += jnp.dot(oh, nf, preferred_element_type=F32)
        c_ref[0] += jnp.sum(oh, axis=1, keepdims=True)

    return pl.pallas_call(
        body,
        grid=(B, P // THW, n // TN),
        in_specs=[pl.BlockSpec((1, TN, C), lambda b, h, t: (b, t, 0)),
                  pl.BlockSpec((1, C), lambda b, h, t: (0, 0)),
                  pl.BlockSpec((1, C), lambda b, h, t: (0, 0)),
                  pl.BlockSpec((1, 16, TN), lambda b, h, t: (b, 0, t))],
        out_specs=[pl.BlockSpec((1, THW, C), lambda b, h, t: (b, h, 0)),
                   pl.BlockSpec((1, THW, 1), lambda b, h, t: (b, h, 0))],
        out_shape=[jax.ShapeDtypeStruct((B, P, C), F32),
                   jax.ShapeDtypeStruct((B, P, 1), F32)],
    )(z2, sc2, sh2, projT)


def kernel(unknown, known, unknow_feats, known_feats, image_features,
           new_vis, V2R, P2, image_shape, mlp_params, f3d_pre, f3d_mlp,
           f2d_mlp, f2d_conv):
    B, n, _ = unknown.shape
    m = known.shape[1]
    Hf, Wf = image_features.shape[2], image_features.shape[3]
    P = Hf * Wf
    C3 = mlp_params[0][0].shape[0]
    C2d = image_features.shape[1]
    R3 = B * n
    R2 = B * P

    uf_t = unknow_feats.transpose(0, 2, 1)
    kf = known_feats.transpose(0, 2, 1)
    known_t = known.transpose(0, 2, 1)
    imf = image_features.transpose(0, 2, 3, 1).reshape(B, P, C2d)

    interp = _knn_interp(unknown, known_t, kf)

    projs = []
    for bs in range(B):
        kp = unknown[bs]
        hom = jnp.concatenate([kp, jnp.ones((n, 1), dtype=kp.dtype)], -1)
        c0 = hom @ V2R[bs].T
        c2 = c0 @ P2[bs].T
        depth = c2[:, 2]
        uv = c2[:, :2] / depth[:, None]
        u = uv[:, 0] * Wf / image_shape[1]
        v = uv[:, 1] * Hf / image_shape[0]
        x0 = jnp.floor(u).astype(jnp.int32)
        y0 = jnp.floor(v).astype(jnp.int32)
        x1 = x0 + 1
        y1 = y0 + 1
        x0c = jnp.clip(x0, 0, Wf - 1); x1c = jnp.clip(x1, 0, Wf - 1)
        y0c = jnp.clip(y0, 0, Hf - 1); y1c = jnp.clip(y1, 0, Hf - 1)
        x0f = x0c.astype(u.dtype); x1f = x1c.astype(u.dtype)
        y0f = y0c.astype(v.dtype); y1f = y1c.astype(v.dtype)
        wa = (x1f - u) * (y1f - v); wb = (x1f - u) * (v - y0f)
        wc = (u - x0f) * (y1f - v); wd = (u - x0f) * (v - y0f)
        qa = (y0c * Wf + x0c).astype(F32)
        qb = (y1c * Wf + x0c).astype(F32)
        qc = (y0c * Wf + x1c).astype(F32)
        qd = (y1c * Wf + x1c).astype(F32)
        vis1 = (new_vis[bs] == 1).astype(F32)
        mask = ((u >= 0) & (u < Wf) & (v >= 0) & (v < Hf)
                & (new_vis[bs] > 0)).astype(F32)
        ug = jnp.clip(jnp.floor(u).astype(jnp.int32), 0, Wf - 1)
        vg = jnp.clip(jnp.floor(v).astype(jnp.int32), 0, Hf - 1)
        s = (vg * Wf + ug).astype(F32)
        zero = jnp.zeros_like(s)
        projs.append(jnp.stack(
            [qa, qb, qc, qd, wa, wb, wc, wd, s, mask, vis1,
             zero, zero, zero, zero, zero], axis=-1))
    proj = jnp.stack(projs)
    projT = proj.transpose(0, 2, 1)

    W1, g1, b1 = mlp_params[0]
    C2 = kf.shape[2]
    z1, st1 = _fused_mm([interp.reshape(R3, C2), uf_t.reshape(R3, -1)],
                        [W1[:, :C2].T, W1[:, C2:].T], [None, None])
    a1 = _bn_affine(st1, g1, b1, R3)
    W2, g2, b2 = mlp_params[1]
    z2, st2 = _fused_mm([z1], [W2.T], [a1])
    a2 = _bn_affine(st2, g2, b2, R3)

    pif = _pif_gather(proj, imf)
    zp = pif.reshape(R3, C2d)
    ap = None
    for (Wp, gp, bp) in f3d_pre:
        zp, stp = _fused_mm([zp], [Wp.T], [ap])
        ap = _bn_affine(stp, gp, bp, R3)

    z2b = z2.reshape(B, n, C3)
    g_grid, cnt = _scatter_grid(z2b, a2[0], a2[1], projT, P)
    zg = g_grid.reshape(R2, C3)
    cg = cnt.reshape(R2, 1)
    Wg, gg, bg = f2d_mlp[0]
    zg, stg = _fused_mm([zg], [Wg.T], [None], cnt=cg)
    ag = _bn_affine(stg, gg, bg, R2)
    for (Wn, gn, bn) in f2d_mlp[1:]:
        zg, stg = _fused_mm([zg], [Wn.T], [ag])
        ag = _bn_affine(stg, gn, bn, R2)
    Wc, gc, bc = f2d_conv
    zc, stc = _fused_mm([zg, imf.reshape(R2, C2d)],
                        [Wc[:, :C3].T, Wc[:, C3:].T], [ag, None])
    ac = _bn_affine(stc, gc, bc, R2)
    out_img = _apply_affine_relu(zc, ac[0], ac[1])
    new_image_features = (out_img.reshape(B, Hf, Wf, C2d)
                          .transpose(0, 3, 1, 2))

    Wf3, gf3, bf3 = f3d_mlp
    zf, stf = _fused_mm([z2, zp], [Wf3[:, :C3].T, Wf3[:, C3:].T], [a2, ap])
    af = _bn_affine(stf, gf3, bf3, R3)
    out_pts = _apply_affine_relu(zf, af[0], af[1])
    new_features = out_pts.reshape(B, n, C3).transpose(0, 2, 1)

    return (new_features, new_image_features)


# speedup vs baseline: 6.7912x; 1.0005x over previous
import functools

import jax
import jax.numpy as jnp
from jax import lax
from jax.experimental import pallas as pl
from jax.experimental.pallas import tpu as pltpu
from jax.experimental.pallas import tpu_sc as plsc

F32 = jnp.float32


def _tile(R, cap=512):
    for t in (cap, 256, 128, 64, 32, 16, 8, 4, 2, 1):
        if t <= cap and R % t == 0:
            return t
    return 1


def _fused_mm(xs, Ws, affs, cnt=None):
    R = xs[0].shape[0]
    Cout = Ws[0].shape[1]
    TR = _tile(R)
    NT = R // TR
    nin = len(xs)
    have_cnt = cnt is not None

    def body(*refs):
        i = 0
        xr = refs[i:i + nin]; i += nin
        wr = refs[i:i + nin]; i += nin
        affr = []
        for a in affs:
            if a is not None:
                affr.append((refs[i], refs[i + 1])); i += 2
            else:
                affr.append(None)
        c_ref = None
        if have_cnt:
            c_ref = refs[i]; i += 1
        z_ref, st_ref = refs[i], refs[i + 1]
        acc = None
        for k in range(nin):
            x = xr[k][...]
            if affr[k] is not None:
                x = jnp.maximum(x * affr[k][0][...] + affr[k][1][...], 0.0)
            if k == 0 and c_ref is not None:
                x = x * (1.0 / (c_ref[...] + 1e-6))
            d = jnp.dot(x, wr[k][...], preferred_element_type=F32)
            acc = d if acc is None else acc + d
        z_ref[...] = acc

        @pl.when(pl.program_id(0) == 0)
        def _():
            st_ref[...] = jnp.zeros_like(st_ref)
        st_ref[0:1, :] += jnp.sum(acc, axis=0, keepdims=True)
        st_ref[1:2, :] += jnp.sum(acc * acc, axis=0, keepdims=True)

    ins, specs = [], []
    for x in xs:
        ins.append(x)
        specs.append(pl.BlockSpec((TR, x.shape[1]), lambda i: (i, 0)))
    for W in Ws:
        ins.append(W)
        specs.append(pl.BlockSpec(W.shape, lambda i: (0, 0)))
    for a in affs:
        if a is not None:
            for v in a:
                ins.append(v)
                specs.append(pl.BlockSpec((1, v.shape[1]), lambda i: (0, 0)))
    if have_cnt:
        ins.append(cnt)
        specs.append(pl.BlockSpec((TR, 1), lambda i: (i, 0)))

    z, st = pl.pallas_call(
        body,
        grid=(NT,),
        in_specs=specs,
        out_specs=[pl.BlockSpec((TR, Cout), lambda i: (i, 0)),
                   pl.BlockSpec((8, Cout), lambda i: (0, 0))],
        out_shape=[jax.ShapeDtypeStruct((R, Cout), F32),
                   jax.ShapeDtypeStruct((8, Cout), F32)],
    )(*ins)
    return z, st


def _bn_affine(st, g, b, R):
    mean = st[0] / R
    var = st[1] / R - mean * mean
    sc = g / jnp.sqrt(var + 1e-5)
    sh = b - mean * sc
    return sc.reshape(1, -1).astype(F32), sh.reshape(1, -1).astype(F32)


def _apply_affine_relu(z, sc, sh, rowmask=None):
    R, C = z.shape
    TR = _tile(R)

    def body(*refs):
        if rowmask is None:
            z_ref, s_ref, h_ref, o_ref = refs
            o_ref[...] = jnp.maximum(
                z_ref[...] * s_ref[...] + h_ref[...], 0.0)
        else:
            z_ref, s_ref, h_ref, m_ref, o_ref = refs
            o_ref[...] = jnp.maximum(
                z_ref[...] * s_ref[...] + h_ref[...], 0.0) * m_ref[...]

    specs = [pl.BlockSpec((TR, C), lambda i: (i, 0)),
             pl.BlockSpec((1, C), lambda i: (0, 0)),
             pl.BlockSpec((1, C), lambda i: (0, 0))]
    ins = [z, sc, sh]
    if rowmask is not None:
        specs.append(pl.BlockSpec((TR, 1), lambda i: (i, 0)))
        ins.append(rowmask)
    return pl.pallas_call(
        body,
        grid=(R // TR,),
        in_specs=specs,
        out_specs=pl.BlockSpec((TR, C), lambda i: (i, 0)),
        out_shape=jax.ShapeDtypeStruct((R, C), F32),
    )(*ins)


def _sc_scatter(nf4, sidx, zf, P):
    _, B, n, Cq = nf4.shape
    NS = 16
    CH = 128
    ppt = n // NS
    nch = ppt // CH
    rpt = P // NS
    mesh = plsc.VectorSubcoreMesh(core_axis_name="c", subcore_axis_name="s")

    @functools.partial(
        pl.kernel, mesh=mesh,
        out_type=jax.ShapeDtypeStruct((4, B, P, Cq), F32),
        scratch_types=[
            pltpu.VMEM((CH,), jnp.int32),
            pltpu.VMEM((CH, Cq), F32),
            pltpu.VMEM((rpt // 2, Cq), F32),
            pltpu.VMEM_SHARED((P, Cq), F32),
        ],
    )
    def k(nf4_h, sidx_h, zf_h, g_h, idx_v, feat_v, row_v, acc_s):
        cid = lax.axis_index("c")
        sid = lax.axis_index("s")
        hrp = rpt // 2
        for b in range(B):
            for q in range(2):
                for hh in range(2):
                    rr = sid * rpt + hh * hrp
                    pltpu.sync_copy(zf_h.at[pl.ds(rr, hrp)], row_v)
                    pltpu.sync_copy(row_v, acc_s.at[pl.ds(rr, hrp)])
                plsc.subcore_barrier()
                for ck in range(nch):
                    base = sid * ppt + ck * CH
                    pltpu.sync_copy(sidx_h.at[b, pl.ds(base, CH)], idx_v)
                    pltpu.sync_copy(nf4_h.at[cid * 2 + q, b,
                                             pl.ds(base, CH)], feat_v)
                    pltpu.sync_copy(feat_v, acc_s.at[idx_v], add=True)
                plsc.subcore_barrier()
                for hh in range(2):
                    rr = sid * rpt + hh * hrp
                    pltpu.sync_copy(acc_s.at[pl.ds(rr, hrp)], row_v)
                    pltpu.sync_copy(row_v, g_h.at[cid * 2 + q, b,
                                                  pl.ds(rr, hrp)])

    return k(nf4, sidx, zf)


def _count_grid(sidxT, maskT, P):
    B, n = sidxT.shape[0], sidxT.shape[2]
    TN = _tile(n)
    THW = _tile(P, 1920)

    def body(s_ref, m_ref, c_ref):
        t = pl.program_id(2)
        h = pl.program_id(1)
        srow = s_ref[0]
        mrow = m_ref[0]
        base = (h * THW).astype(F32)
        iop = jax.lax.broadcasted_iota(jnp.int32, (THW, TN), 0).astype(F32) + base
        oh = jnp.where(iop == srow, mrow, 0.0)

        @pl.when(t == 0)
        def _():
            c_ref[...] = jnp.zeros_like(c_ref)
        c_ref[0] += jnp.sum(oh, axis=1, keepdims=True)

    return pl.pallas_call(
        body,
        grid=(B, P // THW, n // TN),
        in_specs=[pl.BlockSpec((1, 1, TN), lambda b, h, t: (b, 0, t)),
                  pl.BlockSpec((1, 1, TN), lambda b, h, t: (b, 0, t))],
        out_specs=pl.BlockSpec((1, THW, 1), lambda b, h, t: (b, h, 0)),
        out_shape=jax.ShapeDtypeStruct((B, P, 1), F32),
    )(sidxT, maskT)


def _knn_interp(unknown, known_t, kf):
    B, n, _ = unknown.shape
    m = known_t.shape[2]
    C = kf.shape[2]
    TN = _tile(n, 256)

    def body(u_ref, k_ref, f_ref, o_ref):
        u = u_ref[0]
        k = k_ref[0]
        d2 = ((u[:, 0:1] - k[0:1, :]) ** 2
              + (u[:, 1:2] - k[1:2, :]) ** 2
              + (u[:, 2:3] - k[2:3, :]) ** 2)
        m1 = jnp.min(d2, axis=1, keepdims=True)
        d2a = jnp.where(d2 > m1, d2, jnp.inf)
        m2 = jnp.min(d2a, axis=1, keepdims=True)
        d2b = jnp.where(d2a > m2, d2a, jnp.inf)
        m3 = jnp.min(d2b, axis=1, keepdims=True)
        msk = d2 <= m3
        dist = jnp.sqrt(jnp.maximum(d2, 1e-12))
        recip = jnp.where(msk, 1.0 / (dist + 1e-8), 0.0)
        wgt = recip / jnp.sum(recip, axis=1, keepdims=True)
        o_ref[0] = jnp.dot(wgt, f_ref[0], preferred_element_type=F32)

    return pl.pallas_call(
        body,
        grid=(B, n // TN),
        in_specs=[pl.BlockSpec((1, TN, 3), lambda b, t: (b, t, 0)),
                  pl.BlockSpec((1, 3, m), lambda b, t: (b, 0, 0)),
                  pl.BlockSpec((1, m, C), lambda b, t: (b, 0, 0))],
        out_specs=pl.BlockSpec((1, TN, C), lambda b, t: (b, t, 0)),
        out_shape=jax.ShapeDtypeStruct((B, n, C), F32),
    )(unknown, known_t, kf)


def _pif_gather(proj, imf):
    B, n, _ = proj.shape
    P, C = imf.shape[1], imf.shape[2]
    TN = _tile(n)
    THW = _tile(P, 1920)
    NHW = P // THW

    def body(p_ref, im_ref, o_ref):
        k = pl.program_id(2)
        base = (k * THW).astype(F32)
        pj = p_ref[0]
        iot = jax.lax.broadcasted_iota(jnp.int32, (TN, THW), 1).astype(F32) + base
        S = (jnp.where(iot == pj[:, 0:1], pj[:, 4:5], 0.0)
             + jnp.where(iot == pj[:, 1:2], pj[:, 5:6], 0.0)
             + jnp.where(iot == pj[:, 2:3], pj[:, 6:7], 0.0)
             + jnp.where(iot == pj[:, 3:4], pj[:, 7:8], 0.0))
        S = S * pj[:, 10:11]
        acc = jnp.dot(S, im_ref[0], preferred_element_type=F32)

        @pl.when(k == 0)
        def _():
            o_ref[...] = jnp.zeros_like(o_ref)
        o_ref[0] += acc

    return pl.pallas_call(
        body,
        grid=(B, n // TN, NHW),
        in_specs=[pl.BlockSpec((1, TN, 16), lambda b, t, k: (b, t, 0)),
                  pl.BlockSpec((1, THW, C), lambda b, t, k: (b, k, 0))],
        out_specs=pl.BlockSpec((1, TN, C), lambda b, t, k: (b, t, 0)),
        out_shape=jax.ShapeDtypeStruct((B, n, C), F32),
    )(proj, imf)


def _sc_scatter(nf4, sidx, zf, P):
    _, B, n, Cq = nf4.shape
    NS = 16
    CH = 128
    ppt = n // NS
    nch = ppt // CH
    rpt = P // NS
    mesh = plsc.VectorSubcoreMesh(core_axis_name="c", subcore_axis_name="s")

    @functools.partial(
        pl.kernel, mesh=mesh,
        out_type=jax.ShapeDtypeStruct((4, B, P, Cq), F32),
        scratch_types=[
            pltpu.VMEM((CH,), jnp.int32),
            pltpu.VMEM((CH, Cq), F32),
            pltpu.VMEM((rpt // 2, Cq), F32),
            pltpu.VMEM_SHARED((P, Cq), F32),
        ],
    )
    def k(nf4_h, sidx_h, zf_h, g_h, idx_v, feat_v, row_v, acc_s):
        cid = lax.axis_index("c")
        sid = lax.axis_index("s")
        hrp = rpt // 2
        for b in range(B):
            for q in range(2):
                for hh in range(2):
                    rr = sid * rpt + hh * hrp
                    pltpu.sync_copy(zf_h.at[pl.ds(rr, hrp)], row_v)
                    pltpu.sync_copy(row_v, acc_s.at[pl.ds(rr, hrp)])
                plsc.subcore_barrier()
                for ck in range(nch):
                    base = sid * ppt + ck * CH
                    pltpu.sync_copy(sidx_h.at[b, pl.ds(base, CH)], idx_v)
                    pltpu.sync_copy(nf4_h.at[cid * 2 + q, b,
                                             pl.ds(base, CH)], feat_v)
                    pltpu.sync_copy(feat_v, acc_s.at[idx_v], add=True)
                plsc.subcore_barrier()
                for hh in range(2):
                    rr = sid * rpt + hh * hrp
                    pltpu.sync_copy(acc_s.at[pl.ds(rr, hrp)], row_v)
                    pltpu.sync_copy(row_v, g_h.at[cid * 2 + q, b,
                                                  pl.ds(rr, hrp)])

    return k(nf4, sidx, zf)


def _count_grid(sidxT, maskT, P):
    B, n = sidxT.shape[0], sidxT.shape[2]
    TN = _tile(n)
    THW = _tile(P, 1920)

    def body(s_ref, m_ref, c_ref):
        t = pl.program_id(2)
        h = pl.program_id(1)
        srow = s_ref[0]
        mrow = m_ref[0]
        base = (h * THW).astype(F32)
        iop = jax.lax.broadcasted_iota(jnp.int32, (THW, TN), 0).astype(F32) + base
        oh = jnp.where(iop == srow, mrow, 0.0)

        @pl.when(t == 0)
        def _():
            c_ref[...] = jnp.zeros_like(c_ref)
        c_ref[0] += jnp.sum(oh, axis=1, keepdims=True)

    return pl.pallas_call(
        body,
        grid=(B, P // THW, n // TN),
        in_specs=[pl.BlockSpec((1, 1, TN), lambda b, h, t: (b, 0, t)),
                  pl.BlockSpec((1, 1, TN), lambda b, h, t: (b, 0, t))],
        out_specs=pl.BlockSpec((1, THW, 1), lambda b, h, t: (b, h, 0)),
        out_shape=jax.ShapeDtypeStruct((B, P, 1), F32),
    )(sidxT, maskT)


def _knn_interp(unknown, known_t, kf):
    B, n, _ = unknown.shape
    m = known_t.shape[2]
    C = kf.shape[2]
    TN = _tile(n, 256)

    def body(u_ref, k_ref, f_ref, o_ref):
        u = u_ref[0]
        k = k_ref[0]
        d2 = ((u[:, 0:1] - k[0:1, :]) ** 2
              + (u[:, 1:2] - k[1:2, :]) ** 2
              + (u[:, 2:3] - k[2:3, :]) ** 2)
        m1 = jnp.min(d2, axis=1, keepdims=True)
        d2a = jnp.where(d2 > m1, d2, jnp.inf)
        m2 = jnp.min(d2a, axis=1, keepdims=True)
        d2b = jnp.where(d2a > m2, d2a, jnp.inf)
        m3 = jnp.min(d2b, axis=1, keepdims=True)
        msk = d2 <= m3
        dist = jnp.sqrt(jnp.maximum(d2, 1e-12))
        recip = jnp.where(msk, 1.0 / (dist + 1e-8), 0.0)
        wgt = recip / jnp.sum(recip, axis=1, keepdims=True)
        o_ref[0] = jnp.dot(wgt, f_ref[0], preferred_element_type=F32)

    return pl.pallas_call(
        body,
        grid=(B, n // TN),
        in_specs=[pl.BlockSpec((1, TN, 3), lambda b, t: (b, t, 0)),
                  pl.BlockSpec((1, 3, m), lambda b, t: (b, 0, 0)),
                  pl.BlockSpec((1, m, C), lambda b, t: (b, 0, 0))],
        out_specs=pl.BlockSpec((1, TN, C), lambda b, t: (b, t, 0)),
        out_shape=jax.ShapeDtypeStruct((B, n, C), F32),
    )(unknown, known_t, kf)


def _pif_gather(proj, imf):
    B, n, _ = proj.shape
    P, C = imf.shape[1], imf.shape[2]
    TN = _tile(n)
    THW = _tile(P, 1920)
    NHW = P // THW

    def body(p_ref, im_ref, o_ref):
        k = pl.program_id(2)
        base = (k * THW).astype(F32)
        pj = p_ref[0]
        iot = jax.lax.broadcasted_iota(jnp.int32, (TN, THW), 1).astype(F32) + base
        S = (jnp.where(iot == pj[:, 0:1], pj[:, 4:5], 0.0)
             + jnp.where(iot == pj[:, 1:2], pj[:, 5:6], 0.0)
             + jnp.where(iot == pj[:, 2:3], pj[:, 6:7], 0.0)
             + jnp.where(iot == pj[:, 3:4], pj[:, 7:8], 0.0))
        S = S * pj[:, 10:11]
        acc = jnp.dot(S, im_ref[0], preferred_element_type=F32)

        @pl.when(k == 0)
        def _():
            o_ref[...] = jnp.zeros_like(o_ref)
        o_ref[0] += acc

    return pl.pallas_call(
        body,
        grid=(B, n // TN, NHW),
        in_specs=[pl.BlockSpec((1, TN, 16), lambda b, t, k: (b, t, 0)),
                  pl.BlockSpec((1, THW, C), lambda b, t, k: (b, k, 0))],
        out_specs=pl.BlockSpec((1, TN, C), lambda b, t, k: (b, t, 0)),
        out_shape=jax.ShapeDtypeStruct((B, n, C), F32),
    )(proj, imf)


def _scatter_grid(z2, sc2, sh2, projT, P):
    B, n, C = z2.shape
    TN = _tile(n)
    THW = _tile(P, 1920)

    def body(z_ref, s_ref, h_ref, pT_ref, g_ref, c_ref):
        t = pl.program_id(2)
        h = pl.program_id(1)
        nf = jnp.maximum(z_ref[0] * s_ref[...] + h_ref[...], 0.0)
        srow = pT_ref[0, 8:9, :]
        mrow = pT_ref[0, 9:10, :]
        base = (h * THW).astype(F32)
        iop = jax.lax.broadcasted_iota(jnp.int32, (THW, TN), 0).astype(F32) + base
        oh = jnp.where(iop == srow, mrow, 0.0)

        @pl.when(t == 0)
        def _():
            g_ref[...] = jnp.zeros_like(g_ref)
            c_ref[...] = jnp.zeros_like(c_ref)
        g_ref[0] += jnp.dot(oh, nf, preferred_element_type=F32)
        c_ref[0] += jnp.sum(oh, axis=1, keepdims=True)

    return pl.pallas_call(
        body,
        grid=(B, P // THW, n // TN),
        in_specs=[pl.BlockSpec((1, TN, C), lambda b, h, t: (b, t, 0)),
                  pl.BlockSpec((1, C), lambda b, h, t: (0, 0)),
                  pl.BlockSpec((1, C), lambda b, h, t: (0, 0)),
                  pl.BlockSpec((1, 16, TN), lambda b, h, t: (b, 0, t))],
        out_specs=[pl.BlockSpec((1, THW, C), lambda b, h, t: (b, h, 0)),
                   pl.BlockSpec((1, THW, 1), lambda b, h, t: (b, h, 0))],
        out_shape=[jax.ShapeDtypeStruct((B, P, C), F32),
                   jax.ShapeDtypeStruct((B, P, 1), F32)],
    )(z2, sc2, sh2, projT)


def kernel(unknown, known, unknow_feats, known_feats, image_features,
           new_vis, V2R, P2, image_shape, mlp_params, f3d_pre, f3d_mlp,
           f2d_mlp, f2d_conv):
    B, n, _ = unknown.shape
    m = known.shape[1]
    Hf, Wf = image_features.shape[2], image_features.shape[3]
    P = Hf * Wf
    C3 = mlp_params[0][0].shape[0]
    C2d = image_features.shape[1]
    R3 = B * n
    R2 = B * P

    uf_t = unknow_feats.transpose(0, 2, 1)
    kf = known_feats.transpose(0, 2, 1)
    known_t = known.transpose(0, 2, 1)
    imf = image_features.transpose(0, 2, 3, 1).reshape(B, P, C2d)

    interp = _knn_interp(unknown, known_t, kf)

    projs, sidx_l, mask_l = [], [], []
    for bs in range(B):
        kp = unknown[bs]
        hom = jnp.concatenate([kp, jnp.ones((n, 1), dtype=kp.dtype)], -1)
        c0 = hom @ V2R[bs].T
        c2 = c0 @ P2[bs].T
        depth = c2[:, 2]
        uv = c2[:, :2] / depth[:, None]
        u = uv[:, 0] * Wf / image_shape[1]
        v = uv[:, 1] * Hf / image_shape[0]
        x0 = jnp.floor(u).astype(jnp.int32)
        y0 = jnp.floor(v).astype(jnp.int32)
        x1 = x0 + 1
        y1 = y0 + 1
        x0c = jnp.clip(x0, 0, Wf - 1); x1c = jnp.clip(x1, 0, Wf - 1)
        y0c = jnp.clip(y0, 0, Hf - 1); y1c = jnp.clip(y1, 0, Hf - 1)
        x0f = x0c.astype(u.dtype); x1f = x1c.astype(u.dtype)
        y0f = y0c.astype(v.dtype); y1f = y1c.astype(v.dtype)
        wa = (x1f - u) * (y1f - v); wb = (x1f - u) * (v - y0f)
        wc = (u - x0f) * (y1f - v); wd = (u - x0f) * (v - y0f)
        qa = (y0c * Wf + x0c).astype(F32)
        qb = (y1c * Wf + x0c).astype(F32)
        qc = (y0c * Wf + x1c).astype(F32)
        qd = (y1c * Wf + x1c).astype(F32)
        vis1 = (new_vis[bs] == 1).astype(F32)
        mask = ((u >= 0) & (u < Wf) & (v >= 0) & (v < Hf)
                & (new_vis[bs] > 0)).astype(F32)
        ug = jnp.clip(jnp.floor(u).astype(jnp.int32), 0, Wf - 1)
        vg = jnp.clip(jnp.floor(v).astype(jnp.int32), 0, Hf - 1)
        s = (vg * Wf + ug).astype(F32)
        zero = jnp.zeros_like(s)
        projs.append(jnp.stack(
            [qa, qb, qc, qd, wa, wb, wc, wd, s, mask, vis1,
             zero, zero, zero, zero, zero], axis=-1))
        sidx_l.append(vg * Wf + ug)
        mask_l.append(mask)
    proj = jnp.stack(projs)
    sidx = jnp.stack(sidx_l)
    maskb = jnp.stack(mask_l)

    W1, g1, b1 = mlp_params[0]
    C2 = kf.shape[2]
    z1, st1 = _fused_mm([interp.reshape(R3, C2), uf_t.reshape(R3, -1)],
                        [W1[:, :C2].T, W1[:, C2:].T], [None, None])
    a1 = _bn_affine(st1, g1, b1, R3)
    W2, g2, b2 = mlp_params[1]
    z2, st2 = _fused_mm([z1], [W2.T], [a1])
    a2 = _bn_affine(st2, g2, b2, R3)

    pif = _pif_gather(proj, imf)
    zp = pif.reshape(R3, C2d)
    ap = None
    for (Wp, gp, bp) in f3d_pre:
        zp, stp = _fused_mm([zp], [Wp.T], [ap])
        ap = _bn_affine(stp, gp, bp, R3)

    nfm = _apply_affine_relu(z2, a2[0], a2[1],
                             rowmask=maskb.reshape(R3, 1))
    nf4 = nfm.reshape(B, n, 4, C3 // 4).transpose(2, 0, 1, 3)
    zf = jnp.zeros((P, C3 // 4), F32)
    g4 = _sc_scatter(nf4, sidx, zf, P)
    zg = jnp.concatenate([g4[0], g4[1], g4[2], g4[3]],
                         axis=-1).reshape(R2, C3)
    cnt = _count_grid(sidx.astype(F32).reshape(B, 1, n),
                      maskb.reshape(B, 1, n), P)
    cg = cnt.reshape(R2, 1)
    Wg, gg, bg = f2d_mlp[0]
    zg, stg = _fused_mm([zg], [Wg.T], [None], cnt=cg)
    ag = _bn_affine(stg, gg, bg, R2)
    for (Wn, gn, bn) in f2d_mlp[1:]:
        zg, stg = _fused_mm([zg], [Wn.T], [ag])
        ag = _bn_affine(stg, gn, bn, R2)
    Wc, gc, bc = f2d_conv
    zc, stc = _fused_mm([zg, imf.reshape(R2, C2d)],
                        [Wc[:, :C3].T, Wc[:, C3:].T], [ag, None])
    ac = _bn_affine(stc, gc, bc, R2)
    out_img = _apply_affine_relu(zc, ac[0], ac[1])
    new_image_features = (out_img.reshape(B, Hf, Wf, C2d)
                          .transpose(0, 3, 1, 2))

    Wf3, gf3, bf3 = f3d_mlp
    zf, stf = _fused_mm([z2, zp], [Wf3[:, :C3].T, Wf3[:, C3:].T], [a2, ap])
    af = _bn_affine(stf, gf3, bf3, R3)
    out_pts = _apply_affine_relu(zf, af[0], af[1])
    new_features = out_pts.reshape(B, n, C3).transpose(0, 2, 1)

    return (new_features, new_image_features)


# speedup vs baseline: 7.9673x; 1.1732x over previous
import functools

import jax
import jax.numpy as jnp
from jax import lax
from jax.experimental import pallas as pl
from jax.experimental.pallas import tpu as pltpu
from jax.experimental.pallas import tpu_sc as plsc

F32 = jnp.float32


def _tile(R, cap=512):
    for t in (cap, 256, 128, 64, 32, 16, 8, 4, 2, 1):
        if t <= cap and R % t == 0:
            return t
    return 1


def _fused_mm(xs, Ws, affs, cnt=None):
    R = xs[0].shape[0]
    Cout = Ws[0].shape[1]
    TR = _tile(R)
    NT = R // TR
    nin = len(xs)
    have_cnt = cnt is not None

    def body(*refs):
        i = 0
        xr = refs[i:i + nin]; i += nin
        wr = refs[i:i + nin]; i += nin
        affr = []
        for a in affs:
            if a is not None:
                affr.append((refs[i], refs[i + 1])); i += 2
            else:
                affr.append(None)
        c_ref = None
        if have_cnt:
            c_ref = refs[i]; i += 1
        z_ref, st_ref = refs[i], refs[i + 1]
        acc = None
        for k in range(nin):
            x = xr[k][...]
            if affr[k] is not None:
                x = jnp.maximum(x * affr[k][0][...] + affr[k][1][...], 0.0)
            if k == 0 and c_ref is not None:
                x = x * (1.0 / (c_ref[...] + 1e-6))
            d = jnp.dot(x, wr[k][...], preferred_element_type=F32)
            acc = d if acc is None else acc + d
        z_ref[...] = acc

        @pl.when(pl.program_id(0) == 0)
        def _():
            st_ref[...] = jnp.zeros_like(st_ref)
        st_ref[0:1, :] += jnp.sum(acc, axis=0, keepdims=True)
        st_ref[1:2, :] += jnp.sum(acc * acc, axis=0, keepdims=True)

    ins, specs = [], []
    for x in xs:
        ins.append(x)
        specs.append(pl.BlockSpec((TR, x.shape[1]), lambda i: (i, 0)))
    for W in Ws:
        ins.append(W)
        specs.append(pl.BlockSpec(W.shape, lambda i: (0, 0)))
    for a in affs:
        if a is not None:
            for v in a:
                ins.append(v)
                specs.append(pl.BlockSpec((1, v.shape[1]), lambda i: (0, 0)))
    if have_cnt:
        ins.append(cnt)
        specs.append(pl.BlockSpec((TR, 1), lambda i: (i, 0)))

    z, st = pl.pallas_call(
        body,
        grid=(NT,),
        in_specs=specs,
        out_specs=[pl.BlockSpec((TR, Cout), lambda i: (i, 0)),
                   pl.BlockSpec((8, Cout), lambda i: (0, 0))],
        out_shape=[jax.ShapeDtypeStruct((R, Cout), F32),
                   jax.ShapeDtypeStruct((8, Cout), F32)],
    )(*ins)
    return z, st


def _bn_affine(st, g, b, R):
    mean = st[0] / R
    var = st[1] / R - mean * mean
    sc = g / jnp.sqrt(var + 1e-5)
    sh = b - mean * sc
    return sc.reshape(1, -1).astype(F32), sh.reshape(1, -1).astype(F32)


def _apply_affine_relu(z, sc, sh, rowmask=None):
    R, C = z.shape
    TR = _tile(R)

    def body(*refs):
        if rowmask is None:
            z_ref, s_ref, h_ref, o_ref = refs
            o_ref[...] = jnp.maximum(
                z_ref[...] * s_ref[...] + h_ref[...], 0.0)
        else:
            z_ref, s_ref, h_ref, m_ref, o_ref = refs
            o_ref[...] = jnp.maximum(
                z_ref[...] * s_ref[...] + h_ref[...], 0.0) * m_ref[...]

    specs = [pl.BlockSpec((TR, C), lambda i: (i, 0)),
             pl.BlockSpec((1, C), lambda i: (0, 0)),
             pl.BlockSpec((1, C), lambda i: (0, 0))]
    ins = [z, sc, sh]
    if rowmask is not None:
        specs.append(pl.BlockSpec((TR, 1), lambda i: (i, 0)))
        ins.append(rowmask)
    return pl.pallas_call(
        body,
        grid=(R // TR,),
        in_specs=specs,
        out_specs=pl.BlockSpec((TR, C), lambda i: (i, 0)),
        out_shape=jax.ShapeDtypeStruct((R, C), F32),
    )(*ins)


def _sc_scatter(nf4, sidx, zf, P):
    _, B, n, Cq = nf4.shape
    NS = 16
    CH = 128
    ppt = n // NS
    nch = ppt // CH
    rpt = P // NS
    mesh = plsc.VectorSubcoreMesh(core_axis_name="c", subcore_axis_name="s")

    @functools.partial(
        pl.kernel, mesh=mesh,
        out_type=jax.ShapeDtypeStruct((4, B, P, Cq), F32),
        scratch_types=[
            pltpu.VMEM((CH,), jnp.int32),
            pltpu.VMEM((CH, Cq), F32),
            pltpu.VMEM((rpt // 2, Cq), F32),
            pltpu.VMEM_SHARED((P, Cq), F32),
        ],
    )
    def k(nf4_h, sidx_h, zf_h, g_h, idx_v, feat_v, row_v, acc_s):
        cid = lax.axis_index("c")
        sid = lax.axis_index("s")
        hrp = rpt // 2
        for b in range(B):
            for q in range(2):
                for hh in range(2):
                    rr = sid * rpt + hh * hrp
                    pltpu.sync_copy(zf_h.at[pl.ds(rr, hrp)], row_v)
                    pltpu.sync_copy(row_v, acc_s.at[pl.ds(rr, hrp)])
                plsc.subcore_barrier()
                for ck in range(nch):
                    base = sid * ppt + ck * CH
                    pltpu.sync_copy(sidx_h.at[b, pl.ds(base, CH)], idx_v)
                    pltpu.sync_copy(nf4_h.at[cid * 2 + q, b,
                                             pl.ds(base, CH)], feat_v)
                    pltpu.sync_copy(feat_v, acc_s.at[idx_v], add=True)
                plsc.subcore_barrier()
                for hh in range(2):
                    rr = sid * rpt + hh * hrp
                    pltpu.sync_copy(acc_s.at[pl.ds(rr, hrp)], row_v)
                    pltpu.sync_copy(row_v, g_h.at[cid * 2 + q, b,
                                                  pl.ds(rr, hrp)])

    return k(nf4, sidx, zf)


def _count_grid(sidxT, maskT, P):
    B, n = sidxT.shape[0], sidxT.shape[2]
    TN = _tile(n)
    THW = _tile(P, 1920)

    def body(s_ref, m_ref, c_ref):
        t = pl.program_id(2)
        h = pl.program_id(1)
        srow = s_ref[0]
        mrow = m_ref[0]
        base = (h * THW).astype(F32)
        iop = jax.lax.broadcasted_iota(
            jnp.int32, (THW, TN), 0).astype(F32) + base
        oh = jnp.where(iop == srow, mrow, 0.0)

        @pl.when(t == 0)
        def _():
            c_ref[...] = jnp.zeros_like(c_ref)
        c_ref[0] += jnp.sum(oh, axis=1, keepdims=True)

    return pl.pallas_call(
        body,
        grid=(B, P // THW, n // TN),
        in_specs=[pl.BlockSpec((1, 1, TN), lambda b, h, t: (b, 0, t)),
                  pl.BlockSpec((1, 1, TN), lambda b, h, t: (b, 0, t))],
        out_specs=pl.BlockSpec((1, THW, 1), lambda b, h, t: (b, h, 0)),
        out_shape=jax.ShapeDtypeStruct((B, P, 1), F32),
    )(sidxT, maskT)


def _knn_interp(unknown, known_t, kf):
    B, n, _ = unknown.shape
    m = known_t.shape[2]
    C = kf.shape[2]
    TN = _tile(n, 256)

    def body(u_ref, k_ref, f_ref, o_ref):
        u = u_ref[0]
        k = k_ref[0]
        d2 = ((u[:, 0:1] - k[0:1, :]) ** 2
              + (u[:, 1:2] - k[1:2, :]) ** 2
              + (u[:, 2:3] - k[2:3, :]) ** 2)
        m1 = jnp.min(d2, axis=1, keepdims=True)
        d2a = jnp.where(d2 > m1, d2, jnp.inf)
        m2 = jnp.min(d2a, axis=1, keepdims=True)
        d2b = jnp.where(d2a > m2, d2a, jnp.inf)
        m3 = jnp.min(d2b, axis=1, keepdims=True)
        msk = d2 <= m3
        dist = jnp.sqrt(jnp.maximum(d2, 1e-12))
        recip = jnp.where(msk, 1.0 / (dist + 1e-8), 0.0)
        wgt = recip / jnp.sum(recip, axis=1, keepdims=True)
        o_ref[0] = jnp.dot(wgt.astype(jnp.bfloat16),
                           f_ref[0].astype(jnp.bfloat16),
                           preferred_element_type=F32)

    return pl.pallas_call(
        body,
        grid=(B, n // TN),
        in_specs=[pl.BlockSpec((1, TN, 3), lambda b, t: (b, t, 0)),
                  pl.BlockSpec((1, 3, m), lambda b, t: (b, 0, 0)),
                  pl.BlockSpec((1, m, C), lambda b, t: (b, 0, 0))],
        out_specs=pl.BlockSpec((1, TN, C), lambda b, t: (b, t, 0)),
        out_shape=jax.ShapeDtypeStruct((B, n, C), F32),
    )(unknown, known_t, kf)


def _sc_gather4(imflat, qidx):
    _, B, n = qidx.shape
    C = imflat.shape[1]
    NW = 32
    CH = 64
    ppw = n // NW
    nch = ppw // CH
    mesh = plsc.VectorSubcoreMesh(core_axis_name="c", subcore_axis_name="s")

    @functools.partial(
        pl.kernel, mesh=mesh,
        out_type=jax.ShapeDtypeStruct((4, B, n, C), F32),
        scratch_types=[
            pltpu.VMEM((CH,), jnp.int32),
            pltpu.VMEM((CH, C), F32),
            pltpu.SemaphoreType.DMA,
        ],
    )
    def k(im_h, q_h, out_h, idx_v, rows_v, sem):
        wid = lax.axis_index("s") * 2 + lax.axis_index("c")
        for b in range(B):
            for c in range(4):
                for ck in range(nch):
                    base = wid * ppw + ck * CH
                    pltpu.sync_copy(q_h.at[c, b, pl.ds(base, CH)], idx_v)
                    pltpu.async_copy(im_h.at[idx_v], rows_v, sem).wait()
                    pltpu.sync_copy(rows_v, out_h.at[c, b, pl.ds(base, CH)])

    return k(imflat, qidx)


def _pif_mm(c0, c1, c2, c3, w4, W):
    R, C = c0.shape
    Cout = W.shape[1]
    TR = _tile(R)

    def body(c0_ref, c1_ref, c2_ref, c3_ref, w_ref, W_ref, z_ref, st_ref):
        pif = (c0_ref[...] * w_ref[:, 0:1] + c1_ref[...] * w_ref[:, 1:2]
               + c2_ref[...] * w_ref[:, 2:3] + c3_ref[...] * w_ref[:, 3:4])
        acc = jnp.dot(pif, W_ref[...], preferred_element_type=F32)
        z_ref[...] = acc

        @pl.when(pl.program_id(0) == 0)
        def _():
            st_ref[...] = jnp.zeros_like(st_ref)
        st_ref[0:1, :] += jnp.sum(acc, axis=0, keepdims=True)
        st_ref[1:2, :] += jnp.sum(acc * acc, axis=0, keepdims=True)

    cs = pl.BlockSpec((TR, C), lambda i: (i, 0))
    return pl.pallas_call(
        body,
        grid=(R // TR,),
        in_specs=[cs, cs, cs, cs,
                  pl.BlockSpec((TR, 4), lambda i: (i, 0)),
                  pl.BlockSpec(W.shape, lambda i: (0, 0))],
        out_specs=[pl.BlockSpec((TR, Cout), lambda i: (i, 0)),
                   pl.BlockSpec((8, Cout), lambda i: (0, 0))],
        out_shape=[jax.ShapeDtypeStruct((R, Cout), F32),
                   jax.ShapeDtypeStruct((8, Cout), F32)],
    )(c0, c1, c2, c3, w4, W)


def kernel(unknown, known, unknow_feats, known_feats, image_features,
           new_vis, V2R, P2, image_shape, mlp_params, f3d_pre, f3d_mlp,
           f2d_mlp, f2d_conv):
    B, n, _ = unknown.shape
    m = known.shape[1]
    Hf, Wf = image_features.shape[2], image_features.shape[3]
    P = Hf * Wf
    C3 = mlp_params[0][0].shape[0]
    C2d = image_features.shape[1]
    R3 = B * n
    R2 = B * P

    uf_t = unknow_feats.transpose(0, 2, 1)
    kf = known_feats.transpose(0, 2, 1)
    known_t = known.transpose(0, 2, 1)
    imf = image_features.transpose(0, 2, 3, 1).reshape(B, P, C2d)

    interp = _knn_interp(unknown, known_t, kf)

    qidx_l, wvis_l, sidx_l, mask_l = [], [], [], []
    for bs in range(B):
        kp = unknown[bs]
        hom = jnp.concatenate([kp, jnp.ones((n, 1), dtype=kp.dtype)], -1)
        c0 = hom @ V2R[bs].T
        c2 = c0 @ P2[bs].T
        depth = c2[:, 2]
        uv = c2[:, :2] / depth[:, None]
        u = uv[:, 0] * Wf / image_shape[1]
        v = uv[:, 1] * Hf / image_shape[0]
        x0 = jnp.floor(u).astype(jnp.int32)
        y0 = jnp.floor(v).astype(jnp.int32)
        x1 = x0 + 1
        y1 = y0 + 1
        x0c = jnp.clip(x0, 0, Wf - 1); x1c = jnp.clip(x1, 0, Wf - 1)
        y0c = jnp.clip(y0, 0, Hf - 1); y1c = jnp.clip(y1, 0, Hf - 1)
        x0f = x0c.astype(u.dtype); x1f = x1c.astype(u.dtype)
        y0f = y0c.astype(v.dtype); y1f = y1c.astype(v.dtype)
        wa = (x1f - u) * (y1f - v); wb = (x1f - u) * (v - y0f)
        wc = (u - x0f) * (y1f - v); wd = (u - x0f) * (v - y0f)
        vis1 = new_vis[bs] == 1
        mask = ((u >= 0) & (u < Wf) & (v >= 0) & (v < Hf)
                & (new_vis[bs] > 0)).astype(F32)
        ug = jnp.clip(jnp.floor(u).astype(jnp.int32), 0, Wf - 1)
        vg = jnp.clip(jnp.floor(v).astype(jnp.int32), 0, Hf - 1)
        qidx_l.append(jnp.stack(
            [y0c * Wf + x0c, y1c * Wf + x0c,
             y0c * Wf + x1c, y1c * Wf + x1c]) + bs * P)
        wvis_l.append(jnp.stack(
            [jnp.where(vis1, wa, 0.0), jnp.where(vis1, wb, 0.0),
             jnp.where(vis1, wc, 0.0), jnp.where(vis1, wd, 0.0)], axis=-1))
        sidx_l.append(vg * Wf + ug)
        mask_l.append(mask)
    qidx = jnp.stack(qidx_l, axis=1)
    wvis = jnp.stack(wvis_l)
    sidx = jnp.stack(sidx_l)
    maskb = jnp.stack(mask_l)

    corners = _sc_gather4(imf.reshape(B * P, C2d), qidx)

    W1, g1, b1 = mlp_params[0]
    C2 = kf.shape[2]
    z1, st1 = _fused_mm([interp.reshape(R3, C2), uf_t.reshape(R3, -1)],
                        [W1[:, :C2].T, W1[:, C2:].T], [None, None])
    a1 = _bn_affine(st1, g1, b1, R3)
    W2, g2, b2 = mlp_params[1]
    z2, st2 = _fused_mm([z1], [W2.T], [a1])
    a2 = _bn_affine(st2, g2, b2, R3)

    Wp0, gp0, bp0 = f3d_pre[0]
    cr = corners.reshape(4, R3, C2d)
    zp, stp = _pif_mm(cr[0], cr[1], cr[2], cr[3],
                      wvis.reshape(R3, 4), Wp0.T)
    ap = _bn_affine(stp, gp0, bp0, R3)
    for (Wp, gp, bp) in f3d_pre[1:]:
        zp, stp = _fused_mm([zp], [Wp.T], [ap])
        ap = _bn_affine(stp, gp, bp, R3)

    nfm = _apply_affine_relu(z2, a2[0], a2[1],
                             rowmask=maskb.reshape(R3, 1))
    nf4 = nfm.reshape(B, n, 4, C3 // 4).transpose(2, 0, 1, 3)
    zf = jnp.zeros((P, C3 // 4), F32)
    g4 = _sc_scatter(nf4, sidx, zf, P)
    zg = jnp.concatenate([g4[0], g4[1], g4[2], g4[3]],
                         axis=-1).reshape(R2, C3)
    cnt = _count_grid(sidx.astype(F32).reshape(B, 1, n),
                      maskb.reshape(B, 1, n), P)
    cg = cnt.reshape(R2, 1)
    Wg, gg, bg = f2d_mlp[0]
    zg, stg = _fused_mm([zg], [Wg.T], [None], cnt=cg)
    ag = _bn_affine(stg, gg, bg, R2)
    for (Wn, gn, bn) in f2d_mlp[1:]:
        zg, stg = _fused_mm([zg], [Wn.T], [ag])
        ag = _bn_affine(stg, gn, bn, R2)
    Wc, gc, bc = f2d_conv
    zc, stc = _fused_mm([zg, imf.reshape(R2, C2d)],
                        [Wc[:, :C3].T, Wc[:, C3:].T], [ag, None])
    ac = _bn_affine(stc, gc, bc, R2)
    out_img = _apply_affine_relu(zc, ac[0], ac[1])
    new_image_features = (out_img.reshape(B, Hf, Wf, C2d)
                          .transpose(0, 3, 1, 2))

    Wf3, gf3, bf3 = f3d_mlp
    zf, stf = _fused_mm([z2, zp], [Wf3[:, :C3].T, Wf3[:, C3:].T], [a2, ap])
    af = _bn_affine(stf, gf3, bf3, R3)
    out_pts = _apply_affine_relu(zf, af[0], af[1])
    new_features = out_pts.reshape(B, n, C3).transpose(0, 2, 1)

    return (new_features, new_image_features)
